# baseline (speedup 1.0000x reference)
import dataclasses
import functools

import jax
import jax.numpy as jnp
from jax import lax
from jax.experimental import pallas as pl
from jax.experimental.pallas import tpu as pltpu
from jax.experimental.pallas import tpu_sc as plsc

N = 10000
NPAD = 10240
E = 320000
F_IN = 128
F_HID = 156
F_OUT = 7
PAD_OUT = 16
NC = 2
NS = 16
NW = NC * NS
RPS = NPAD // NS


def _sc_mesh():
    return plsc.VectorSubcoreMesh(core_axis_name="c", subcore_axis_name="s")


def _sc_params():
    cp = pltpu.CompilerParams()
    fields = pltpu.CompilerParams.__dataclass_fields__
    if "needs_layout_passes" in fields:
        cp = dataclasses.replace(cp, needs_layout_passes=False)
    if "use_tc_tiling_on_sc" in fields:
        cp = dataclasses.replace(cp, use_tc_tiling_on_sc=False)
    return cp


def _zero_fill(ref, nrows, d):
    @pl.loop(0, nrows)
    def _(i):
        @pl.loop(0, d // 16)
        def _(j):
            ref[i, pl.ds(j * 16, 16)] = jnp.zeros((16,), jnp.float32)


def _load_my_idx(idx2d_hbm, idx_v, wid, base_ch, extra_w):
    row0 = base_ch * wid + jnp.minimum(wid, extra_w)
    pltpu.sync_copy(idx2d_hbm.at[pl.ds(row0, base_ch)],
                    idx_v.at[pl.ds(0, base_ch)])

    @pl.when(wid < extra_w)
    def _():
        pltpu.sync_copy(idx2d_hbm.at[pl.ds(row0 + base_ch, 1)],
                        idx_v.at[pl.ds(base_ch, 1)])


def _deg_sc(dst2d):
    chunk = 128
    nchunk = E // chunk
    base_ch = nchunk // NW
    extra_w = nchunk - base_ch * NW
    npair = base_ch // 2

    @functools.partial(
        pl.kernel,
        out_type=jax.ShapeDtypeStruct((NC, NPAD, PAD_OUT), jnp.float32),
        mesh=_sc_mesh(),
        scratch_types=[
            pltpu.VMEM((base_ch + 1, chunk), jnp.int32),
            pltpu.VMEM((chunk, PAD_OUT), jnp.float32),
            pltpu.SemaphoreType.DMA,
            pltpu.SemaphoreType.DMA,
            pltpu.VMEM_SHARED((NPAD, PAD_OUT), jnp.float32),
        ],
        compiler_params=_sc_params(),
    )
    def deg_kernel(dst_hbm, out_hbm, didx_v, ones_v, sems0, sems1, acc_sh):
        cid = lax.axis_index("c")
        sid = lax.axis_index("s")
        wid = sid * NC + cid
        sems = [sems0, sems1]

        _zero_fill(ones_v, chunk, PAD_OUT)
        for j in range(RPS // chunk):
            pltpu.sync_copy(ones_v, acc_sh.at[pl.ds(sid * RPS + j * chunk, chunk)])

        @pl.loop(0, chunk)
        def _(i):
            ones_v[i, :] = jnp.full((PAD_OUT,), 1.0, jnp.float32)

        _load_my_idx(dst_hbm, didx_v, wid, base_ch, extra_w)
        plsc.subcore_barrier()

        def scat_start(j, ss):
            pltpu.async_copy(ones_v, acc_sh.at[didx_v.at[j]], sems[ss], add=True)

        def scat_wait(ss):
            pltpu.make_async_copy(ones_v, acc_sh.at[didx_v.at[0]],
                                  sems[ss]).wait()

        scat_start(0, 0)
        scat_start(1, 1)

        @pl.loop(1, npair)
        def _(p):
            scat_wait(0)
            scat_start(2 * p, 0)
            scat_wait(1)
            scat_start(2 * p + 1, 1)

        scat_wait(0)

        @pl.when(wid < extra_w)
        def _():
            scat_start(base_ch, 0)

        scat_wait(1)

        @pl.when(wid < extra_w)
        def _():
            scat_wait(0)

        plsc.subcore_barrier()
        pltpu.sync_copy(
            acc_sh.at[pl.ds(sid * RPS, RPS)],
            out_hbm.at[cid, pl.ds(sid * RPS, RPS)],
        )

    return deg_kernel(dst2d)


def _agg_sc(y, src2d, dst2d, d, chunk, stage):
    nchunk = E // chunk
    base_ch = nchunk // NW
    extra_w = nchunk - base_ch * NW
    npair = base_ch // 2

    scratch = [
        pltpu.VMEM((base_ch + 1, chunk), jnp.int32),
        pltpu.VMEM((base_ch + 1, chunk), jnp.int32),
        pltpu.VMEM((chunk, d), jnp.float32),
        pltpu.VMEM((chunk, d), jnp.float32),
        pltpu.SemaphoreType.DMA,
        pltpu.SemaphoreType.DMA,
        pltpu.SemaphoreType.DMA,
        pltpu.SemaphoreType.DMA,
        pltpu.VMEM_SHARED((NPAD, d), jnp.float32),
    ]
    if stage:
        scratch.append(pltpu.VMEM_SHARED((NPAD, d), jnp.float32))

    @functools.partial(
        pl.kernel,
        out_type=jax.ShapeDtypeStruct((NC, NPAD, d), jnp.float32),
        mesh=_sc_mesh(),
        scratch_types=scratch,
        compiler_params=_sc_params(),
    )
    def agg_kernel(y_hbm, src_hbm, dst_hbm, out_hbm,
                   sidx_v, didx_v, rows0, rows1,
                   semg0, semg1, sems0, sems1, acc_sh, *maybe_ysp):
        cid = lax.axis_index("c")
        sid = lax.axis_index("s")
        wid = sid * NC + cid
        rows = [rows0, rows1]
        semg = [semg0, semg1]
        sems = [sems0, sems1]
        y_src = maybe_ysp[0] if stage else y_hbm

        _zero_fill(rows0, chunk, d)
        for j in range(RPS // chunk):
            pltpu.sync_copy(rows0, acc_sh.at[pl.ds(sid * RPS + j * chunk, chunk)])

        if stage:
            pltpu.sync_copy(y_hbm.at[pl.ds(sid * RPS, RPS)],
                            maybe_ysp[0].at[pl.ds(sid * RPS, RPS)])

        _load_my_idx(src_hbm, sidx_v, wid, base_ch, extra_w)
        _load_my_idx(dst_hbm, didx_v, wid, base_ch, extra_w)
        plsc.subcore_barrier()

        def gath_start(j, rs):
            pltpu.async_copy(y_src.at[sidx_v.at[j]], rows[rs], semg[rs])

        def gath_wait(rs):
            pltpu.make_async_copy(y_src.at[sidx_v.at[0]], rows[rs],
                                  semg[rs]).wait()

        def scat_start(j, rs):
            pltpu.async_copy(rows[rs], acc_sh.at[didx_v.at[j]], sems[rs],
                             add=True)

        def scat_wait(rs):
            pltpu.make_async_copy(rows[0], acc_sh.at[didx_v.at[0]],
                                  sems[rs]).wait()

        gath_start(0, 0)
        gath_start(1, 1)
        gath_wait(0)
        scat_start(0, 0)
        gath_wait(1)
        scat_start(1, 1)

        @pl.loop(1, npair)
        def _(p):
            scat_wait(0)
            gath_start(2 * p, 0)
            scat_wait(1)
            gath_start(2 * p + 1, 1)
            gath_wait(0)
            scat_start(2 * p, 0)
            gath_wait(1)
            scat_start(2 * p + 1, 1)

        scat_wait(0)

        @pl.when(wid < extra_w)
        def _():
            gath_start(base_ch, 0)
            gath_wait(0)
            scat_start(base_ch, 0)

        scat_wait(1)

        @pl.when(wid < extra_w)
        def _():
            scat_wait(0)

        plsc.subcore_barrier()
        pltpu.sync_copy(
            acc_sh.at[pl.ds(sid * RPS, RPS)],
            out_hbm.at[cid, pl.ds(sid * RPS, RPS)],
        )

    return agg_kernel(y, src2d, dst2d)


def _agg_sc_wide(y1a, y1b, src2d, dst2d):
    d = F_IN // 2
    chunk = 128
    nchunk = E // chunk
    base_ch = nchunk // NW
    extra_w = nchunk - base_ch * NW
    npair = base_ch // 2

    @functools.partial(
        pl.kernel,
        out_type=jax.ShapeDtypeStruct((2, NC, NPAD, d), jnp.float32),
        mesh=_sc_mesh(),
        scratch_types=[
            pltpu.VMEM((base_ch + 1, chunk), jnp.int32),
            pltpu.VMEM((base_ch + 1, chunk), jnp.int32),
            pltpu.VMEM((chunk, d), jnp.float32),
            pltpu.VMEM((chunk, d), jnp.float32),
            pltpu.SemaphoreType.DMA,
            pltpu.SemaphoreType.DMA,
            pltpu.SemaphoreType.DMA,
            pltpu.SemaphoreType.DMA,
            pltpu.VMEM_SHARED((NPAD, d), jnp.float32),
            pltpu.VMEM_SHARED((NPAD, d), jnp.float32),
        ],
        compiler_params=_sc_params(),
    )
    def aggw_kernel(ya_hbm, yb_hbm, src_hbm, dst_hbm, out_hbm,
                    sidx_v, didx_v, rows0, rows1,
                    semg0, semg1, sems0, sems1, acc_sh, y_sp):
        cid = lax.axis_index("c")
        sid = lax.axis_index("s")
        wid = sid * NC + cid
        rows = [rows0, rows1]
        semg = [semg0, semg1]
        sems = [sems0, sems1]

        _load_my_idx(src_hbm, sidx_v, wid, base_ch, extra_w)
        _load_my_idx(dst_hbm, didx_v, wid, base_ch, extra_w)

        def gath_start(j, rs):
            pltpu.async_copy(y_sp.at[sidx_v.at[j]], rows[rs], semg[rs])

        def gath_wait(rs):
            pltpu.make_async_copy(y_sp.at[sidx_v.at[0]], rows[rs],
                                  semg[rs]).wait()

        def scat_start(j, rs):
            pltpu.async_copy(rows[rs], acc_sh.at[didx_v.at[j]], sems[rs],
                             add=True)

        def scat_wait(rs):
            pltpu.make_async_copy(rows[0], acc_sh.at[didx_v.at[0]],
                                  sems[rs]).wait()

        for h, y_hbm in enumerate((ya_hbm, yb_hbm)):
            _zero_fill(rows0, chunk, d)
            for j in range(RPS // chunk):
                pltpu.sync_copy(rows0,
                                acc_sh.at[pl.ds(sid * RPS + j * chunk, chunk)])
            pltpu.sync_copy(y_hbm.at[pl.ds(sid * RPS, RPS)],
                            y_sp.at[pl.ds(sid * RPS, RPS)])
            plsc.subcore_barrier()

            gath_start(0, 0)
            gath_start(1, 1)
            gath_wait(0)
            scat_start(0, 0)
            gath_wait(1)
            scat_start(1, 1)

            @pl.loop(1, npair)
            def _(p):
                scat_wait(0)
                gath_start(2 * p, 0)
                scat_wait(1)
                gath_start(2 * p + 1, 1)
                gath_wait(0)
                scat_start(2 * p, 0)
                gath_wait(1)
                scat_start(2 * p + 1, 1)

            scat_wait(0)

            @pl.when(wid < extra_w)
            def _():
                gath_start(base_ch, 0)
                gath_wait(0)
                scat_start(base_ch, 0)

            scat_wait(1)

            @pl.when(wid < extra_w)
            def _():
                scat_wait(0)

            plsc.subcore_barrier()
            pltpu.sync_copy(
                acc_sh.at[pl.ds(sid * RPS, RPS)],
                out_hbm.at[h, cid, pl.ds(sid * RPS, RPS)],
            )
            plsc.subcore_barrier()

    return aggw_kernel(y1a, y1b, src2d, dst2d)


def _dinv_from(deg_r):
    deg = deg_r[0, :, 0:1] + deg_r[1, :, 0:1] + 1.0
    return lax.rsqrt(deg)


def _tc_prescale(deg, x):
    H = F_IN // 2

    def body(d_r, x_ref, ya_ref, yb_ref):
        dinv = _dinv_from(d_r)[0:N]
        ya_ref[0:N, :] = x_ref[:, 0:H] * dinv
        yb_ref[0:N, :] = x_ref[:, H:F_IN] * dinv
        ya_ref[N:NPAD, :] = jnp.zeros((NPAD - N, H), jnp.float32)
        yb_ref[N:NPAD, :] = jnp.zeros((NPAD - N, H), jnp.float32)

    return pl.pallas_call(
        body,
        out_shape=(jax.ShapeDtypeStruct((NPAD, H), jnp.float32),
                   jax.ShapeDtypeStruct((NPAD, H), jnp.float32)),
    )(deg, x)


def _tc_mid(S1, y1a, y1b, deg, W1, b1, W2p):
    H = F_IN // 2

    def body(s_r, y1a_r, y1b_r, d_r, w1_r, b1_r, w2_r, y2_r):
        dinv = _dinv_from(d_r)
        ua = (s_r[0, 0] + s_r[0, 1] + y1a_r[...]) * dinv
        ub = (s_r[1, 0] + s_r[1, 1] + y1b_r[...]) * dinv
        h = (jnp.dot(ua, w1_r[0:H, :], preferred_element_type=jnp.float32)
             + jnp.dot(ub, w1_r[H:F_IN, :], preferred_element_type=jnp.float32)
             + b1_r[...])
        h = jnp.maximum(h, 0.0)
        z = jnp.dot(h, w2_r[...], preferred_element_type=jnp.float32)
        y2_r[...] = z * dinv

    B = NPAD // 4
    return pl.pallas_call(
        body,
        grid=(NPAD // B,),
        in_specs=[
            pl.BlockSpec((2, 2, B, H), lambda i: (0, 0, i, 0)),
            pl.BlockSpec((B, H), lambda i: (i, 0)),
            pl.BlockSpec((B, H), lambda i: (i, 0)),
            pl.BlockSpec((2, B, PAD_OUT), lambda i: (0, i, 0)),
            pl.BlockSpec((F_IN, F_HID), lambda i: (0, 0)),
            pl.BlockSpec((1, F_HID), lambda i: (0, 0)),
            pl.BlockSpec((F_HID, PAD_OUT), lambda i: (0, 0)),
        ],
        out_specs=pl.BlockSpec((B, PAD_OUT), lambda i: (i, 0)),
        out_shape=jax.ShapeDtypeStruct((NPAD, PAD_OUT), jnp.float32),
    )(S1, y1a, y1b, deg, W1, b1, W2p)


def _tc_post(S2, y2, deg, b2):
    def body(s_r, y2_r, d_r, b2_r, o_r):
        dinv = _dinv_from(d_r)[0:N]
        v = (s_r[0, 0:N, :] + s_r[1, 0:N, :] + y2_r[0:N, :]) * dinv
        logits = v[:, 0:F_OUT] + b2_r[...]
        m = jnp.max(logits, axis=1, keepdims=True)
        sh = logits - m
        lse = jnp.log(jnp.sum(jnp.exp(sh), axis=1, keepdims=True))
        o_r[...] = sh - lse

    return pl.pallas_call(
        body, out_shape=jax.ShapeDtypeStruct((N, F_OUT), jnp.float32)
    )(S2, y2, deg, b2)


def kernel(x, edges, W1, b1, W2, b2):
    src = edges[0].astype(jnp.int32)
    dst = edges[1].astype(jnp.int32)
    W2p = jnp.zeros((F_HID, PAD_OUT), jnp.float32).at[:, :F_OUT].set(W2)
    b1r = b1.reshape(1, F_HID)
    b2r = b2.reshape(1, F_OUT)

    deg = _deg_sc(dst.reshape(E // 128, 128))
    y1a, y1b = _tc_prescale(deg, x)
    S1 = _agg_sc_wide(y1a, y1b, src.reshape(E // 128, 128),
                      dst.reshape(E // 128, 128))
    y2 = _tc_mid(S1, y1a, y1b, deg, W1, b1r, W2p)
    S2 = _agg_sc(y2, src.reshape(E // 128, 128), dst.reshape(E // 128, 128),
                 PAD_OUT, 128, stage=True)
    return _tc_post(S2, y2, deg, b2r)


# speedup vs baseline: 1.2358x; 1.2358x over previous
import dataclasses
import functools

import jax
import jax.numpy as jnp
from jax import lax
from jax.experimental import pallas as pl
from jax.experimental.pallas import tpu as pltpu
from jax.experimental.pallas import tpu_sc as plsc

N = 10000
NPAD = 10240
E = 320000
F_IN = 128
F_HID = 156
F_OUT = 7
PAD_OUT = 16
NC = 2
NS = 16
NW = NC * NS
RPS = NPAD // NS


def _sc_mesh():
    return plsc.VectorSubcoreMesh(core_axis_name="c", subcore_axis_name="s")


def _sc_params():
    cp = pltpu.CompilerParams()
    fields = pltpu.CompilerParams.__dataclass_fields__
    if "needs_layout_passes" in fields:
        cp = dataclasses.replace(cp, needs_layout_passes=False)
    if "use_tc_tiling_on_sc" in fields:
        cp = dataclasses.replace(cp, use_tc_tiling_on_sc=False)
    return cp


def _zero_fill(ref, nrows, d):
    @pl.loop(0, nrows)
    def _(i):
        @pl.loop(0, d // 16)
        def _(j):
            ref[i, pl.ds(j * 16, 16)] = jnp.zeros((16,), jnp.float32)


def _load_my_idx(idx2d_hbm, idx_v, wid, base_ch, extra_w):
    row0 = base_ch * wid + jnp.minimum(wid, extra_w)
    pltpu.sync_copy(idx2d_hbm.at[pl.ds(row0, base_ch)],
                    idx_v.at[pl.ds(0, base_ch)])

    @pl.when(wid < extra_w)
    def _():
        pltpu.sync_copy(idx2d_hbm.at[pl.ds(row0 + base_ch, 1)],
                        idx_v.at[pl.ds(base_ch, 1)])


def _deg_sc(dst2d):
    chunk = 128
    nchunk = E // chunk
    base_ch = nchunk // NW
    extra_w = nchunk - base_ch * NW
    npair = base_ch // 2

    @functools.partial(
        pl.kernel,
        out_type=jax.ShapeDtypeStruct((NC, NPAD, PAD_OUT), jnp.float32),
        mesh=_sc_mesh(),
        scratch_types=[
            pltpu.VMEM((base_ch + 1, chunk), jnp.int32),
            pltpu.VMEM((chunk, PAD_OUT), jnp.float32),
            pltpu.SemaphoreType.DMA,
            pltpu.SemaphoreType.DMA,
            pltpu.VMEM_SHARED((NPAD, PAD_OUT), jnp.float32),
        ],
        compiler_params=_sc_params(),
    )
    def deg_kernel(dst_hbm, out_hbm, didx_v, ones_v, sems0, sems1, acc_sh):
        cid = lax.axis_index("c")
        sid = lax.axis_index("s")
        wid = sid * NC + cid
        sems = [sems0, sems1]

        _zero_fill(ones_v, chunk, PAD_OUT)
        for j in range(RPS // chunk):
            pltpu.sync_copy(ones_v, acc_sh.at[pl.ds(sid * RPS + j * chunk, chunk)])

        @pl.loop(0, chunk)
        def _(i):
            ones_v[i, :] = jnp.full((PAD_OUT,), 1.0, jnp.float32)

        _load_my_idx(dst_hbm, didx_v, wid, base_ch, extra_w)
        plsc.subcore_barrier()

        def scat_start(j, ss):
            pltpu.async_copy(ones_v, acc_sh.at[didx_v.at[j]], sems[ss], add=True)

        def scat_wait(ss):
            pltpu.make_async_copy(ones_v, acc_sh.at[didx_v.at[0]],
                                  sems[ss]).wait()

        scat_start(0, 0)
        scat_start(1, 1)

        @pl.loop(1, npair)
        def _(p):
            scat_wait(0)
            scat_start(2 * p, 0)
            scat_wait(1)
            scat_start(2 * p + 1, 1)

        scat_wait(0)

        @pl.when(wid < extra_w)
        def _():
            scat_start(base_ch, 0)

        scat_wait(1)

        @pl.when(wid < extra_w)
        def _():
            scat_wait(0)

        plsc.subcore_barrier()
        pltpu.sync_copy(
            acc_sh.at[pl.ds(sid * RPS, RPS)],
            out_hbm.at[cid, pl.ds(sid * RPS, RPS)],
        )

    return deg_kernel(dst2d)


def _agg_sc(y, src2d, dst2d, d, chunk, stage):
    nchunk = E // chunk
    base_ch = nchunk // NW
    extra_w = nchunk - base_ch * NW
    npair = base_ch // 2

    scratch = [
        pltpu.VMEM((base_ch + 1, chunk), jnp.int32),
        pltpu.VMEM((base_ch + 1, chunk), jnp.int32),
        pltpu.VMEM((chunk, d), jnp.float32),
        pltpu.VMEM((chunk, d), jnp.float32),
        pltpu.SemaphoreType.DMA,
        pltpu.SemaphoreType.DMA,
        pltpu.SemaphoreType.DMA,
        pltpu.SemaphoreType.DMA,
        pltpu.VMEM_SHARED((NPAD, d), jnp.float32),
    ]
    if stage:
        scratch.append(pltpu.VMEM_SHARED((NPAD, d), jnp.float32))

    @functools.partial(
        pl.kernel,
        out_type=jax.ShapeDtypeStruct((NC, NPAD, d), jnp.float32),
        mesh=_sc_mesh(),
        scratch_types=scratch,
        compiler_params=_sc_params(),
    )
    def agg_kernel(y_hbm, src_hbm, dst_hbm, out_hbm,
                   sidx_v, didx_v, rows0, rows1,
                   semg0, semg1, sems0, sems1, acc_sh, *maybe_ysp):
        cid = lax.axis_index("c")
        sid = lax.axis_index("s")
        wid = sid * NC + cid
        rows = [rows0, rows1]
        semg = [semg0, semg1]
        sems = [sems0, sems1]
        y_src = maybe_ysp[0] if stage else y_hbm

        _zero_fill(rows0, chunk, d)
        for j in range(RPS // chunk):
            pltpu.sync_copy(rows0, acc_sh.at[pl.ds(sid * RPS + j * chunk, chunk)])

        if stage:
            pltpu.sync_copy(y_hbm.at[pl.ds(sid * RPS, RPS)],
                            maybe_ysp[0].at[pl.ds(sid * RPS, RPS)])

        _load_my_idx(src_hbm, sidx_v, wid, base_ch, extra_w)
        _load_my_idx(dst_hbm, didx_v, wid, base_ch, extra_w)
        plsc.subcore_barrier()

        def gath_start(j, rs):
            pltpu.async_copy(y_src.at[sidx_v.at[j]], rows[rs], semg[rs])

        def gath_wait(rs):
            pltpu.make_async_copy(y_src.at[sidx_v.at[0]], rows[rs],
                                  semg[rs]).wait()

        def scat_start(j, rs):
            pltpu.async_copy(rows[rs], acc_sh.at[didx_v.at[j]], sems[rs],
                             add=True)

        def scat_wait(rs):
            pltpu.make_async_copy(rows[0], acc_sh.at[didx_v.at[0]],
                                  sems[rs]).wait()

        gath_start(0, 0)
        gath_start(1, 1)
        gath_wait(0)
        scat_start(0, 0)
        gath_wait(1)
        scat_start(1, 1)

        @pl.loop(1, npair)
        def _(p):
            scat_wait(0)
            gath_start(2 * p, 0)
            scat_wait(1)
            gath_start(2 * p + 1, 1)
            gath_wait(0)
            scat_start(2 * p, 0)
            gath_wait(1)
            scat_start(2 * p + 1, 1)

        scat_wait(0)

        @pl.when(wid < extra_w)
        def _():
            gath_start(base_ch, 0)
            gath_wait(0)
            scat_start(base_ch, 0)

        scat_wait(1)

        @pl.when(wid < extra_w)
        def _():
            scat_wait(0)

        plsc.subcore_barrier()
        pltpu.sync_copy(
            acc_sh.at[pl.ds(sid * RPS, RPS)],
            out_hbm.at[cid, pl.ds(sid * RPS, RPS)],
        )

    return agg_kernel(y, src2d, dst2d)


def _agg_sc3(y, src2d, dst2d, d, chunk):
    nchunk = E // chunk
    base_ch = nchunk // NW
    extra_w = nchunk - base_ch * NW
    ngrp = base_ch // 3
    assert ngrp * 3 == base_ch

    @functools.partial(
        pl.kernel,
        out_type=jax.ShapeDtypeStruct((NC, NPAD, d), jnp.float32),
        mesh=_sc_mesh(),
        scratch_types=[
            pltpu.VMEM((base_ch + 1, chunk), jnp.int32),
            pltpu.VMEM((base_ch + 1, chunk), jnp.int32),
            pltpu.VMEM((chunk, d), jnp.float32),
            pltpu.VMEM((chunk, d), jnp.float32),
            pltpu.VMEM((chunk, d), jnp.float32),
            pltpu.SemaphoreType.DMA,
            pltpu.SemaphoreType.DMA,
            pltpu.SemaphoreType.DMA,
            pltpu.SemaphoreType.DMA,
            pltpu.SemaphoreType.DMA,
            pltpu.SemaphoreType.DMA,
            pltpu.VMEM_SHARED((NPAD, d), jnp.float32),
        ],
        compiler_params=_sc_params(),
    )
    def agg3_kernel(y_hbm, src_hbm, dst_hbm, out_hbm,
                    sidx_v, didx_v, rows0, rows1, rows2,
                    semg0, semg1, semg2, sems0, sems1, sems2, acc_sh):
        cid = lax.axis_index("c")
        sid = lax.axis_index("s")
        wid = sid * NC + cid
        rows = [rows0, rows1, rows2]
        semg = [semg0, semg1, semg2]
        sems = [sems0, sems1, sems2]

        _zero_fill(rows0, chunk, d)
        for j in range(RPS // chunk):
            pltpu.sync_copy(rows0, acc_sh.at[pl.ds(sid * RPS + j * chunk, chunk)])

        _load_my_idx(src_hbm, sidx_v, wid, base_ch, extra_w)
        _load_my_idx(dst_hbm, didx_v, wid, base_ch, extra_w)
        plsc.subcore_barrier()

        def gath_start(j, rs):
            pltpu.async_copy(y_hbm.at[sidx_v.at[j]], rows[rs], semg[rs])

        def gath_wait(rs):
            pltpu.make_async_copy(y_hbm.at[sidx_v.at[0]], rows[rs],
                                  semg[rs]).wait()

        def scat_start(j, rs):
            pltpu.async_copy(rows[rs], acc_sh.at[didx_v.at[j]], sems[rs],
                             add=True)

        def scat_wait(rs):
            pltpu.make_async_copy(rows[0], acc_sh.at[didx_v.at[0]],
                                  sems[rs]).wait()

        for b in range(3):
            gath_start(b, b)

        @pl.loop(0, ngrp - 1)
        def _(p):
            for b in range(3):
                gath_wait(b)
                scat_start(3 * p + b, b)
            for b in range(3):
                scat_wait(b)
                gath_start(3 * p + 3 + b, b)

        for b in range(3):
            gath_wait(b)
            scat_start(3 * (ngrp - 1) + b, b)

        scat_wait(0)

        @pl.when(wid < extra_w)
        def _():
            gath_start(base_ch, 0)
            gath_wait(0)
            scat_start(base_ch, 0)

        scat_wait(1)
        scat_wait(2)

        @pl.when(wid < extra_w)
        def _():
            scat_wait(0)

        plsc.subcore_barrier()
        pltpu.sync_copy(
            acc_sh.at[pl.ds(sid * RPS, RPS)],
            out_hbm.at[cid, pl.ds(sid * RPS, RPS)],
        )

    return agg3_kernel(y, src2d, dst2d)


def _dinv_from(deg_r):
    deg = deg_r[0, :, 0:1] + deg_r[1, :, 0:1] + 1.0
    return lax.rsqrt(deg)


def _tc_prescale(deg, x):
    def body(d_r, x_ref, y_ref):
        y_ref[0:N, :] = x_ref[...] * _dinv_from(d_r)[0:N]
        y_ref[N:NPAD, :] = jnp.zeros((NPAD - N, F_IN), jnp.float32)

    return pl.pallas_call(
        body, out_shape=jax.ShapeDtypeStruct((NPAD, F_IN), jnp.float32)
    )(deg, x)


def _tc_mid(S1, y1, deg, W1, b1, W2p):
    def body(s_r, y1_r, d_r, w1_r, b1_r, w2_r, y2_r):
        dinv = _dinv_from(d_r)
        u = (s_r[0] + s_r[1] + y1_r[...]) * dinv
        h = jnp.dot(u, w1_r[...], preferred_element_type=jnp.float32) + b1_r[...]
        h = jnp.maximum(h, 0.0)
        z = jnp.dot(h, w2_r[...], preferred_element_type=jnp.float32)
        y2_r[...] = z * dinv

    B = NPAD // 4
    return pl.pallas_call(
        body,
        grid=(NPAD // B,),
        in_specs=[
            pl.BlockSpec((2, B, F_IN), lambda i: (0, i, 0)),
            pl.BlockSpec((B, F_IN), lambda i: (i, 0)),
            pl.BlockSpec((2, B, PAD_OUT), lambda i: (0, i, 0)),
            pl.BlockSpec((F_IN, F_HID), lambda i: (0, 0)),
            pl.BlockSpec((1, F_HID), lambda i: (0, 0)),
            pl.BlockSpec((F_HID, PAD_OUT), lambda i: (0, 0)),
        ],
        out_specs=pl.BlockSpec((B, PAD_OUT), lambda i: (i, 0)),
        out_shape=jax.ShapeDtypeStruct((NPAD, PAD_OUT), jnp.float32),
    )(S1, y1, deg, W1, b1, W2p)


def _tc_post(S2, y2, deg, b2):
    def body(s_r, y2_r, d_r, b2_r, o_r):
        dinv = _dinv_from(d_r)[0:N]
        v = (s_r[0, 0:N, :] + s_r[1, 0:N, :] + y2_r[0:N, :]) * dinv
        logits = v[:, 0:F_OUT] + b2_r[...]
        m = jnp.max(logits, axis=1, keepdims=True)
        sh = logits - m
        lse = jnp.log(jnp.sum(jnp.exp(sh), axis=1, keepdims=True))
        o_r[...] = sh - lse

    return pl.pallas_call(
        body, out_shape=jax.ShapeDtypeStruct((N, F_OUT), jnp.float32)
    )(S2, y2, deg, b2)


def kernel(x, edges, W1, b1, W2, b2):
    src = edges[0].astype(jnp.int32)
    dst = edges[1].astype(jnp.int32)
    W2p = jnp.zeros((F_HID, PAD_OUT), jnp.float32).at[:, :F_OUT].set(W2)
    b1r = b1.reshape(1, F_HID)
    b2r = b2.reshape(1, F_OUT)

    deg = _deg_sc(dst.reshape(E // 128, 128))
    y1 = _tc_prescale(deg, x)
    S1 = _agg_sc3(y1, src.reshape(E // 64, 64), dst.reshape(E // 64, 64),
                  F_IN, 64)
    y2 = _tc_mid(S1, y1, deg, W1, b1r, W2p)
    S2 = _agg_sc(y2, src.reshape(E // 128, 128), dst.reshape(E // 128, 128),
                 PAD_OUT, 128, stage=True)
    return _tc_post(S2, y2, deg, b2r)


# speedup vs baseline: 1.2626x; 1.0217x over previous
import dataclasses
import functools

import jax
import jax.numpy as jnp
from jax import lax
from jax.experimental import pallas as pl
from jax.experimental.pallas import tpu as pltpu
from jax.experimental.pallas import tpu_sc as plsc

N = 10000
NPAD = 10240
E = 320000
F_IN = 128
F_HID = 156
F_OUT = 7
PAD_OUT = 16
NC = 2
NS = 16
NW = NC * NS
RPS = NPAD // NS


def _sc_mesh():
    return plsc.VectorSubcoreMesh(core_axis_name="c", subcore_axis_name="s")


def _sc_params():
    cp = pltpu.CompilerParams()
    fields = pltpu.CompilerParams.__dataclass_fields__
    if "needs_layout_passes" in fields:
        cp = dataclasses.replace(cp, needs_layout_passes=False)
    if "use_tc_tiling_on_sc" in fields:
        cp = dataclasses.replace(cp, use_tc_tiling_on_sc=False)
    return cp


def _zero_fill(ref, nrows, d):
    @pl.loop(0, nrows)
    def _(i):
        @pl.loop(0, d // 16)
        def _(j):
            ref[i, pl.ds(j * 16, 16)] = jnp.zeros((16,), jnp.float32)


def _load_my_idx(idx2d_hbm, idx_v, wid, base_ch, extra_w):
    row0 = base_ch * wid + jnp.minimum(wid, extra_w)
    pltpu.sync_copy(idx2d_hbm.at[pl.ds(row0, base_ch)],
                    idx_v.at[pl.ds(0, base_ch)])

    @pl.when(wid < extra_w)
    def _():
        pltpu.sync_copy(idx2d_hbm.at[pl.ds(row0 + base_ch, 1)],
                        idx_v.at[pl.ds(base_ch, 1)])


def _deg_sc(dst2d):
    chunk = 128
    nchunk = E // chunk
    base_ch = nchunk // NW
    extra_w = nchunk - base_ch * NW
    ngrp = base_ch // 3
    assert ngrp * 3 == base_ch

    @functools.partial(
        pl.kernel,
        out_type=jax.ShapeDtypeStruct((NC, NPAD, PAD_OUT), jnp.float32),
        mesh=_sc_mesh(),
        scratch_types=[
            pltpu.VMEM((base_ch + 1, chunk), jnp.int32),
            pltpu.VMEM((chunk, PAD_OUT), jnp.float32),
            pltpu.SemaphoreType.DMA,
            pltpu.SemaphoreType.DMA,
            pltpu.SemaphoreType.DMA,
            pltpu.VMEM_SHARED((NPAD, PAD_OUT), jnp.float32),
        ],
        compiler_params=_sc_params(),
    )
    def deg_kernel(dst_hbm, out_hbm, didx_v, ones_v, sems0, sems1, sems2,
                   acc_sh):
        cid = lax.axis_index("c")
        sid = lax.axis_index("s")
        wid = sid * NC + cid
        sems = [sems0, sems1, sems2]

        _zero_fill(ones_v, chunk, PAD_OUT)
        for j in range(RPS // chunk):
            pltpu.sync_copy(ones_v, acc_sh.at[pl.ds(sid * RPS + j * chunk, chunk)])

        @pl.loop(0, chunk)
        def _(i):
            ones_v[i, :] = jnp.full((PAD_OUT,), 1.0, jnp.float32)

        _load_my_idx(dst_hbm, didx_v, wid, base_ch, extra_w)
        plsc.subcore_barrier()

        def scat_start(j, ss):
            pltpu.async_copy(ones_v, acc_sh.at[didx_v.at[j]], sems[ss], add=True)

        def scat_wait(ss):
            pltpu.make_async_copy(ones_v, acc_sh.at[didx_v.at[0]],
                                  sems[ss]).wait()

        for b in range(3):
            scat_start(b, b)

        @pl.loop(1, ngrp)
        def _(p):
            for b in range(3):
                scat_wait(b)
                scat_start(3 * p + b, b)

        scat_wait(0)

        @pl.when(wid < extra_w)
        def _():
            scat_start(base_ch, 0)

        scat_wait(1)
        scat_wait(2)

        @pl.when(wid < extra_w)
        def _():
            scat_wait(0)

        plsc.subcore_barrier()
        pltpu.sync_copy(
            acc_sh.at[pl.ds(sid * RPS, RPS)],
            out_hbm.at[cid, pl.ds(sid * RPS, RPS)],
        )

    return deg_kernel(dst2d)


def _agg_sc3(y, src2d, dst2d, d, chunk, nbuf, stage):
    nchunk = E // chunk
    base_ch = nchunk // NW
    extra_w = nchunk - base_ch * NW
    ngrp = base_ch // nbuf
    assert ngrp * nbuf == base_ch

    scratch = (
        [pltpu.VMEM((base_ch + 1, chunk), jnp.int32)] * 2
        + [pltpu.VMEM((chunk, d), jnp.float32)] * nbuf
        + [pltpu.SemaphoreType.DMA] * (2 * nbuf)
        + [pltpu.VMEM_SHARED((NPAD, d), jnp.float32)] * (2 if stage else 1)
    )

    @functools.partial(
        pl.kernel,
        out_type=jax.ShapeDtypeStruct((NC, NPAD, d), jnp.float32),
        mesh=_sc_mesh(),
        scratch_types=scratch,
        compiler_params=_sc_params(),
    )
    def agg3_kernel(y_hbm, src_hbm, dst_hbm, out_hbm, sidx_v, didx_v, *rest):
        cid = lax.axis_index("c")
        sid = lax.axis_index("s")
        wid = sid * NC + cid
        rows = list(rest[0:nbuf])
        semg = list(rest[nbuf:2 * nbuf])
        sems = list(rest[2 * nbuf:3 * nbuf])
        acc_sh = rest[3 * nbuf]
        y_src = rest[3 * nbuf + 1] if stage else y_hbm

        _zero_fill(rows[0], chunk, d)
        for j in range(RPS // chunk):
            pltpu.sync_copy(rows[0],
                            acc_sh.at[pl.ds(sid * RPS + j * chunk, chunk)])

        if stage:
            pltpu.sync_copy(y_hbm.at[pl.ds(sid * RPS, RPS)],
                            y_src.at[pl.ds(sid * RPS, RPS)])

        _load_my_idx(src_hbm, sidx_v, wid, base_ch, extra_w)
        _load_my_idx(dst_hbm, didx_v, wid, base_ch, extra_w)
        plsc.subcore_barrier()

        def gath_start(j, rs):
            pltpu.async_copy(y_src.at[sidx_v.at[j]], rows[rs], semg[rs])

        def gath_wait(rs):
            pltpu.make_async_copy(y_src.at[sidx_v.at[0]], rows[rs],
                                  semg[rs]).wait()

        def scat_start(j, rs):
            pltpu.async_copy(rows[rs], acc_sh.at[didx_v.at[j]], sems[rs],
                             add=True)

        def scat_wait(rs):
            pltpu.make_async_copy(rows[0], acc_sh.at[didx_v.at[0]],
                                  sems[rs]).wait()

        for b in range(nbuf):
            gath_start(b, b)

        @pl.loop(0, ngrp - 1)
        def _(p):
            for b in range(nbuf):
                gath_wait(b)
                scat_start(nbuf * p + b, b)
            for b in range(nbuf):
                scat_wait(b)
                gath_start(nbuf * p + nbuf + b, b)

        for b in range(nbuf):
            gath_wait(b)
            scat_start(nbuf * (ngrp - 1) + b, b)

        scat_wait(0)

        @pl.when(wid < extra_w)
        def _():
            gath_start(base_ch, 0)
            gath_wait(0)
            scat_start(base_ch, 0)

        for b in range(1, nbuf):
            scat_wait(b)

        @pl.when(wid < extra_w)
        def _():
            scat_wait(0)

        plsc.subcore_barrier()
        pltpu.sync_copy(
            acc_sh.at[pl.ds(sid * RPS, RPS)],
            out_hbm.at[cid, pl.ds(sid * RPS, RPS)],
        )

    return agg3_kernel(y, src2d, dst2d)


def _dinv_from(deg_r):
    deg = deg_r[0, :, 0:1] + deg_r[1, :, 0:1] + 1.0
    return lax.rsqrt(deg)


def _tc_prescale(deg, x):
    def body(d_r, x_ref, y_ref):
        y_ref[0:N, :] = x_ref[...] * _dinv_from(d_r)[0:N]
        y_ref[N:NPAD, :] = jnp.zeros((NPAD - N, F_IN), jnp.float32)

    return pl.pallas_call(
        body, out_shape=jax.ShapeDtypeStruct((NPAD, F_IN), jnp.float32)
    )(deg, x)


def _tc_mid(S1, y1, deg, W1, b1, W2p):
    def body(s_r, y1_r, d_r, w1_r, b1_r, w2_r, y2_r):
        dinv = _dinv_from(d_r)
        u = (s_r[0] + s_r[1] + y1_r[...]) * dinv
        h = jnp.dot(u, w1_r[...], preferred_element_type=jnp.float32) + b1_r[...]
        h = jnp.maximum(h, 0.0)
        z = jnp.dot(h, w2_r[...], preferred_element_type=jnp.float32)
        y2_r[...] = z * dinv

    B = NPAD // 4
    return pl.pallas_call(
        body,
        grid=(NPAD // B,),
        in_specs=[
            pl.BlockSpec((2, B, F_IN), lambda i: (0, i, 0)),
            pl.BlockSpec((B, F_IN), lambda i: (i, 0)),
            pl.BlockSpec((2, B, PAD_OUT), lambda i: (0, i, 0)),
            pl.BlockSpec((F_IN, F_HID), lambda i: (0, 0)),
            pl.BlockSpec((1, F_HID), lambda i: (0, 0)),
            pl.BlockSpec((F_HID, PAD_OUT), lambda i: (0, 0)),
        ],
        out_specs=pl.BlockSpec((B, PAD_OUT), lambda i: (i, 0)),
        out_shape=jax.ShapeDtypeStruct((NPAD, PAD_OUT), jnp.float32),
    )(S1, y1, deg, W1, b1, W2p)


def _tc_post(S2, y2, deg, b2):
    def body(s_r, y2_r, d_r, b2_r, o_r):
        dinv = _dinv_from(d_r)[0:N]
        v = (s_r[0, 0:N, :] + s_r[1, 0:N, :] + y2_r[0:N, :]) * dinv
        logits = v[:, 0:F_OUT] + b2_r[...]
        m = jnp.max(logits, axis=1, keepdims=True)
        sh = logits - m
        lse = jnp.log(jnp.sum(jnp.exp(sh), axis=1, keepdims=True))
        o_r[...] = sh - lse

    return pl.pallas_call(
        body, out_shape=jax.ShapeDtypeStruct((N, F_OUT), jnp.float32)
    )(S2, y2, deg, b2)


def kernel(x, edges, W1, b1, W2, b2):
    src = edges[0].astype(jnp.int32)
    dst = edges[1].astype(jnp.int32)
    W2p = jnp.zeros((F_HID, PAD_OUT), jnp.float32).at[:, :F_OUT].set(W2)
    b1r = b1.reshape(1, F_HID)
    b2r = b2.reshape(1, F_OUT)

    deg = _deg_sc(dst.reshape(E // 128, 128))
    y1 = _tc_prescale(deg, x)
    S1 = _agg_sc3(y1, src.reshape(E // 64, 64), dst.reshape(E // 64, 64),
                  F_IN, 64, 3, stage=False)
    y2 = _tc_mid(S1, y1, deg, W1, b1r, W2p)
    S2 = _agg_sc3(y2, src.reshape(E // 128, 128), dst.reshape(E // 128, 128),
                  PAD_OUT, 128, 3, stage=True)
    return _tc_post(S2, y2, deg, b2r)
